# baseline (speedup 1.0000x reference)
import functools

import jax
import jax.numpy as jnp
from jax import lax
from jax.experimental import pallas as pl
from jax.experimental.pallas import tpu as pltpu
from jax.experimental.pallas import tpu_sc as plsc

N = 10000
E = 320000
D = 128
U = 128

NC = 2
NS = 16
NW = NC * NS
LANES = 16

EPW = E // NW
K = 128
FULL_CHUNKS = EPW // K
TAIL = EPW - FULL_CHUNKS * K
CH = 2000
RPT = N // NS

_mesh = plsc.VectorSubcoreMesh(core_axis_name="c", subcore_axis_name="s")
_sc_params = pltpu.CompilerParams(needs_layout_passes=False,
                                  use_tc_tiling_on_sc=False)


@functools.partial(
    pl.kernel,
    out_type=(
        jax.ShapeDtypeStruct((NW, N), jnp.float32),
        jax.ShapeDtypeStruct((NW, N), jnp.float32),
    ),
    mesh=_mesh,
    scratch_types=[
        pltpu.VMEM((N,), jnp.float32),
        pltpu.VMEM((N,), jnp.float32),
        [pltpu.VMEM((CH,), jnp.int32) for _ in range(2)],
        [pltpu.VMEM((CH,), jnp.int32) for _ in range(2)],
        [pltpu.SemaphoreType.DMA for _ in range(2)],
    ],
    compiler_params=_sc_params,
)
def _deg_kernel(refA_hbm, refB_hbm, outA, outB, histA, histB, bufA, bufB,
                dsem):
    c = lax.axis_index("c")
    s = lax.axis_index("s")
    wid = s * NC + c

    zeros = jnp.zeros((LANES,), jnp.float32)

    def zero_body(i, carry):
        histA[pl.ds(i * LANES, LANES)] = zeros
        histB[pl.ds(i * LANES, LANES)] = zeros
        return carry

    lax.fori_loop(0, N // LANES, zero_body, 0)

    base = wid * EPW
    ones = jnp.ones((LANES,), jnp.float32)
    NCHUNK = EPW // CH
    UNROLL = 5

    def load(o, b):
        pltpu.async_copy(refA_hbm.at[pl.ds(base + o * CH, CH)], bufA[b],
                         dsem[b])
        pltpu.async_copy(refB_hbm.at[pl.ds(base + o * CH, CH)], bufB[b],
                         dsem[b])

    load(0, 0)
    for o in range(NCHUNK):
        b = o % 2
        if o + 1 < NCHUNK:
            load(o + 1, 1 - b)
        pltpu.make_async_copy(refA_hbm.at[pl.ds(0, CH)], bufA[b],
                              dsem[b]).wait()
        pltpu.make_async_copy(refB_hbm.at[pl.ds(0, CH)], bufB[b],
                              dsem[b]).wait()

        def inner(i, icarry):
            for u in range(UNROLL):
                off = (i * UNROLL + u) * LANES
                ia = bufA[b][pl.ds(off, LANES)]
                ib = bufB[b][pl.ds(off, LANES)]
                plsc.addupdate_scatter(histA, [ia], ones)
                plsc.addupdate_scatter(histB, [ib], ones)
            return icarry

        lax.fori_loop(0, CH // (LANES * UNROLL), inner, 0)

    pltpu.sync_copy(histA, outA.at[wid])
    pltpu.sync_copy(histB, outB.at[wid])


NBUF = 2

@functools.partial(
    pl.kernel,
    out_type=jax.ShapeDtypeStruct((NC * N, D), jnp.float32),
    mesh=_mesh,
    scratch_types=[
        pltpu.VMEM((EPW,), jnp.int32),
        [pltpu.VMEM((K,), jnp.int32) for _ in range(NBUF)],
        pltpu.VMEM((TAIL,), jnp.int32),
        [pltpu.VMEM((K, D), jnp.float32) for _ in range(NBUF)],
        pltpu.VMEM_SHARED((N, D), jnp.float32),
        [pltpu.SemaphoreType.DMA for _ in range(NBUF)],
        [pltpu.SemaphoreType.DMA for _ in range(NBUF)],
        pltpu.SemaphoreType.DMA,
        pltpu.SemaphoreType.DMA,
    ],
    compiler_params=_sc_params,
)
def _agg_kernel(xn_hbm, refA_hbm, refB_hbm, out_hbm,
                idxA_full, idxB_s, idxB_t, rows, acc, gsem, isem, zsem, asem):
    c = lax.axis_index("c")
    s = lax.axis_index("s")

    ebase = (c * NS + s) * EPW
    zeros = jnp.zeros((LANES,), jnp.float32)

    def zrow(r, carry):
        for j in range(D // LANES):
            rows[0][r, pl.ds(j * LANES, LANES)] = zeros
        return carry

    lax.fori_loop(0, K, zrow, 0)

    NZ = RPT // K
    ZREM = RPT - NZ * K
    for j in range(NZ):
        pltpu.async_copy(rows[0].at[pl.ds(0, K)],
                         acc.at[pl.ds(s * RPT + j * K, K)], zsem)
    pltpu.async_copy(rows[0].at[pl.ds(0, ZREM)],
                     acc.at[pl.ds(s * RPT + NZ * K, ZREM)], zsem)
    pltpu.async_copy(refA_hbm.at[pl.ds(ebase, EPW)], idxA_full, asem)

    def load_idxB(o, b):
        pltpu.async_copy(refB_hbm.at[pl.ds(ebase + o * K, K)], idxB_s[b],
                         isem[b])

    for b in range(NBUF):
        load_idxB(b, b)

    for j in range(NZ):
        pltpu.make_async_copy(xn_hbm.at[pl.ds(0, K)], rows[0].at[pl.ds(0, K)],
                              zsem).wait()
    pltpu.make_async_copy(xn_hbm.at[pl.ds(0, ZREM)],
                          rows[0].at[pl.ds(0, ZREM)], zsem).wait()
    plsc.subcore_barrier()
    pltpu.make_async_copy(refA_hbm.at[pl.ds(0, EPW)], idxA_full, asem).wait()

    def start_gather(o, b):
        pltpu.async_copy(xn_hbm.at[idxA_full.at[pl.ds(o * K, K)]],
                         rows[b], gsem[b])

    for b in range(NBUF):
        start_gather(b, b)

    def process(o, b):
        pltpu.make_async_copy(xn_hbm.at[pl.ds(0, K)], rows[b], gsem[b]).wait()
        pltpu.make_async_copy(refB_hbm.at[pl.ds(0, K)], idxB_s[b],
                              isem[b]).wait()
        pltpu.sync_copy(rows[b], acc.at[idxB_s[b]], add=True)

    def group(g, carry):
        for b in range(NBUF):
            o = g * NBUF + b
            process(o, b)
            load_idxB(o + NBUF, b)
            start_gather(o + NBUF, b)
        return carry

    G = (FULL_CHUNKS - NBUF - 1) // NBUF
    lax.fori_loop(0, G, group, 0)
    tail_b = (FULL_CHUNKS - 2) % NBUF
    for o in range(G * NBUF, FULL_CHUNKS):
        b = o % NBUF
        process(o, b)
        if o + NBUF < FULL_CHUNKS:
            load_idxB(o + NBUF, b)
            start_gather(o + NBUF, b)
        if o == FULL_CHUNKS - 2:
            pltpu.async_copy(
                refB_hbm.at[pl.ds(ebase + FULL_CHUNKS * K, TAIL)], idxB_t,
                isem[tail_b])
            pltpu.async_copy(
                xn_hbm.at[idxA_full.at[pl.ds(FULL_CHUNKS * K, TAIL)]],
                rows[tail_b].at[pl.ds(0, TAIL)], gsem[tail_b])

    pltpu.make_async_copy(xn_hbm.at[pl.ds(0, TAIL)],
                          rows[tail_b].at[pl.ds(0, TAIL)],
                          gsem[tail_b]).wait()
    pltpu.make_async_copy(refB_hbm.at[pl.ds(0, TAIL)], idxB_t,
                          isem[tail_b]).wait()
    pltpu.sync_copy(rows[tail_b].at[pl.ds(0, TAIL)], acc.at[idxB_t], add=True)

    plsc.subcore_barrier()
    pltpu.sync_copy(acc.at[pl.ds(s * RPT, RPT)],
                    out_hbm.at[pl.ds(c * N + s * RPT, RPT)])


def _r_body(degA_ref, degB_ref, rA_ref, rB_ref):
    rA_ref[...] = lax.rsqrt(
        jnp.maximum(jnp.sum(degA_ref[...], axis=0), 1.0)).reshape(N, 1)
    rB_ref[...] = lax.rsqrt(
        jnp.maximum(jnp.sum(degB_ref[...], axis=0), 1.0)).reshape(N, 1)


_r_call = pl.pallas_call(
    _r_body,
    out_shape=(jax.ShapeDtypeStruct((N, 1), jnp.float32),
               jax.ShapeDtypeStruct((N, 1), jnp.float32)))


BLK = 1000

def _xn_body(x_ref, r_ref, o_ref):
    o_ref[...] = x_ref[...] * r_ref[...]


_xn_call = pl.pallas_call(
    _xn_body,
    grid=(N // BLK,),
    in_specs=[pl.BlockSpec((BLK, D), lambda i: (i, 0)),
              pl.BlockSpec((BLK, 1), lambda i: (i, 0))],
    out_specs=pl.BlockSpec((BLK, D), lambda i: (i, 0)),
    out_shape=jax.ShapeDtypeStruct((N, D), jnp.float32))


def _out_body(a0_ref, a1_ref, r_ref, w_ref, b_ref, o_ref):
    S = (a0_ref[...] + a1_ref[...]) * r_ref[...]
    y = jnp.dot(S, w_ref[...], preferred_element_type=jnp.float32)
    o_ref[...] = jnp.maximum(y + b_ref[...], 0.0)


_out_call = pl.pallas_call(
    _out_body,
    grid=(N // BLK,),
    in_specs=[pl.BlockSpec((BLK, D), lambda i: (i, 0)),
              pl.BlockSpec((BLK, D), lambda i: (i + N // BLK, 0)),
              pl.BlockSpec((BLK, 1), lambda i: (i, 0)),
              pl.BlockSpec((D, U), lambda i: (0, 0)),
              pl.BlockSpec((1, U), lambda i: (0, 0))],
    out_specs=pl.BlockSpec((BLK, U), lambda i: (i, 0)),
    out_shape=jax.ShapeDtypeStruct((N, U), jnp.float32))


def kernel(X, ref_A, ref_B, W, b):
    degA_parts, degB_parts = _deg_kernel(ref_A, ref_B)
    rA, rB = _r_call(degA_parts, degB_parts)
    Xn = _xn_call(X, rA)
    acc = _agg_kernel(Xn, ref_A, ref_B)
    return _out_call(acc, acc, rB, W, b.reshape(1, U))


# speedup vs baseline: 1.0496x; 1.0496x over previous
import functools

import jax
import jax.numpy as jnp
from jax import lax
from jax.experimental import pallas as pl
from jax.experimental.pallas import tpu as pltpu
from jax.experimental.pallas import tpu_sc as plsc

N = 10000
E = 320000
D = 128
U = 128

NC = 2
NS = 16
NW = NC * NS
LANES = 16

EPW = E // NW
K = 128
FULL_CHUNKS = EPW // K
TAIL = EPW - FULL_CHUNKS * K
CH = 2000
RPT = N // NS

_mesh = plsc.VectorSubcoreMesh(core_axis_name="c", subcore_axis_name="s")
_sc_params = pltpu.CompilerParams(needs_layout_passes=False,
                                  use_tc_tiling_on_sc=False)


@functools.partial(
    pl.kernel,
    out_type=(
        jax.ShapeDtypeStruct((NW, N), jnp.float32),
        jax.ShapeDtypeStruct((NW, N), jnp.float32),
    ),
    mesh=_mesh,
    scratch_types=[
        pltpu.VMEM((N,), jnp.float32),
        pltpu.VMEM((N,), jnp.float32),
        [pltpu.VMEM((CH,), jnp.int32) for _ in range(2)],
        [pltpu.VMEM((CH,), jnp.int32) for _ in range(2)],
        [pltpu.SemaphoreType.DMA for _ in range(2)],
    ],
    compiler_params=_sc_params,
)
def _deg_kernel(refA_hbm, refB_hbm, outA, outB, histA, histB, bufA, bufB,
                dsem):
    c = lax.axis_index("c")
    s = lax.axis_index("s")
    wid = s * NC + c

    zeros = jnp.zeros((LANES,), jnp.float32)

    def zero_body(i, carry):
        histA[pl.ds(i * LANES, LANES)] = zeros
        histB[pl.ds(i * LANES, LANES)] = zeros
        return carry

    lax.fori_loop(0, N // LANES, zero_body, 0)

    base = wid * EPW
    ones = jnp.ones((LANES,), jnp.float32)
    NCHUNK = EPW // CH
    UNROLL = 5

    def load(o, b):
        pltpu.async_copy(refA_hbm.at[pl.ds(base + o * CH, CH)], bufA[b],
                         dsem[b])
        pltpu.async_copy(refB_hbm.at[pl.ds(base + o * CH, CH)], bufB[b],
                         dsem[b])

    load(0, 0)
    for o in range(NCHUNK):
        b = o % 2
        if o + 1 < NCHUNK:
            load(o + 1, 1 - b)
        pltpu.make_async_copy(refA_hbm.at[pl.ds(0, CH)], bufA[b],
                              dsem[b]).wait()
        pltpu.make_async_copy(refB_hbm.at[pl.ds(0, CH)], bufB[b],
                              dsem[b]).wait()

        def inner(i, icarry):
            for u in range(UNROLL):
                off = (i * UNROLL + u) * LANES
                ia = bufA[b][pl.ds(off, LANES)]
                ib = bufB[b][pl.ds(off, LANES)]
                plsc.addupdate_scatter(histA, [ia], ones)
                plsc.addupdate_scatter(histB, [ib], ones)
            return icarry

        lax.fori_loop(0, CH // (LANES * UNROLL), inner, 0)

    pltpu.sync_copy(histA, outA.at[wid])
    pltpu.sync_copy(histB, outB.at[wid])


NBUF = 4

@functools.partial(
    pl.kernel,
    out_type=jax.ShapeDtypeStruct((2 * NC * N, D), jnp.bfloat16),
    mesh=_mesh,
    scratch_types=[
        pltpu.VMEM((EPW,), jnp.int32),
        [pltpu.VMEM((K,), jnp.int32) for _ in range(NBUF)],
        pltpu.VMEM((TAIL,), jnp.int32),
        [pltpu.VMEM((K, D), jnp.bfloat16) for _ in range(NBUF)],
        [pltpu.VMEM_SHARED((N, D), jnp.bfloat16) for _ in range(2)],
        [pltpu.SemaphoreType.DMA for _ in range(NBUF)],
        [pltpu.SemaphoreType.DMA for _ in range(NBUF)],
        pltpu.SemaphoreType.DMA,
        pltpu.SemaphoreType.DMA,
    ],
    compiler_params=_sc_params,
)
def _agg_kernel(xn_hbm, refA_hbm, refB_hbm, out_hbm,
                idxA_full, idxB_s, idxB_t, rows, accs, gsem, isem, zsem, asem):
    c = lax.axis_index("c")
    s = lax.axis_index("s")

    ebase = (c * NS + s) * EPW
    zeros = jnp.zeros((2 * LANES,), jnp.bfloat16)

    def zrow(r, carry):
        for j in range(D // (2 * LANES)):
            rows[0][r, pl.ds(j * 2 * LANES, 2 * LANES)] = zeros
        return carry

    lax.fori_loop(0, K, zrow, 0)

    NZ = RPT // K
    ZREM = RPT - NZ * K
    for acc in accs:
        for j in range(NZ):
            pltpu.async_copy(rows[0].at[pl.ds(0, K)],
                             acc.at[pl.ds(s * RPT + j * K, K)], zsem)
        pltpu.async_copy(rows[0].at[pl.ds(0, ZREM)],
                         acc.at[pl.ds(s * RPT + NZ * K, ZREM)], zsem)
    pltpu.async_copy(refA_hbm.at[pl.ds(ebase, EPW)], idxA_full, asem)

    def load_idxB(o, b):
        pltpu.async_copy(refB_hbm.at[pl.ds(ebase + o * K, K)], idxB_s[b],
                         isem[b])

    for b in range(NBUF):
        load_idxB(b, b)

    for _ in range(2):
        for j in range(NZ):
            pltpu.make_async_copy(xn_hbm.at[pl.ds(0, K)],
                                  rows[0].at[pl.ds(0, K)], zsem).wait()
        pltpu.make_async_copy(xn_hbm.at[pl.ds(0, ZREM)],
                              rows[0].at[pl.ds(0, ZREM)], zsem).wait()
    plsc.subcore_barrier()
    pltpu.make_async_copy(refA_hbm.at[pl.ds(0, EPW)], idxA_full, asem).wait()

    def start_gather(o, b):
        pltpu.async_copy(xn_hbm.at[idxA_full.at[pl.ds(o * K, K)]],
                         rows[b], gsem[b])

    for b in range(NBUF):
        start_gather(b, b)

    def process(o, b):
        pltpu.make_async_copy(xn_hbm.at[pl.ds(0, K)], rows[b], gsem[b]).wait()
        pltpu.make_async_copy(refB_hbm.at[pl.ds(0, K)], idxB_s[b],
                              isem[b]).wait()
        pltpu.sync_copy(rows[b], accs[b % 2].at[idxB_s[b]], add=True)

    def group(g, carry):
        for b in range(NBUF):
            o = g * NBUF + b
            process(o, b)
            load_idxB(o + NBUF, b)
            start_gather(o + NBUF, b)
        return carry

    G = (FULL_CHUNKS - NBUF - 1) // NBUF
    lax.fori_loop(0, G, group, 0)
    tail_b = (FULL_CHUNKS - 2) % NBUF
    for o in range(G * NBUF, FULL_CHUNKS):
        b = o % NBUF
        process(o, b)
        if o + NBUF < FULL_CHUNKS:
            load_idxB(o + NBUF, b)
            start_gather(o + NBUF, b)
        if o == FULL_CHUNKS - 2:
            pltpu.async_copy(
                refB_hbm.at[pl.ds(ebase + FULL_CHUNKS * K, TAIL)], idxB_t,
                isem[tail_b])
            pltpu.async_copy(
                xn_hbm.at[idxA_full.at[pl.ds(FULL_CHUNKS * K, TAIL)]],
                rows[tail_b].at[pl.ds(0, TAIL)], gsem[tail_b])

    pltpu.make_async_copy(xn_hbm.at[pl.ds(0, TAIL)],
                          rows[tail_b].at[pl.ds(0, TAIL)],
                          gsem[tail_b]).wait()
    pltpu.make_async_copy(refB_hbm.at[pl.ds(0, TAIL)], idxB_t,
                          isem[tail_b]).wait()
    pltpu.sync_copy(rows[tail_b].at[pl.ds(0, TAIL)], accs[0].at[idxB_t],
                    add=True)

    plsc.subcore_barrier()
    for p in range(2):
        pltpu.sync_copy(
            accs[p].at[pl.ds(s * RPT, RPT)],
            out_hbm.at[pl.ds((2 * c + p) * N + s * RPT, RPT)])


def _xn_body(degA_ref, x_ref, o_ref):
    deg = jnp.sum(degA_ref[...], axis=0)
    r = lax.rsqrt(jnp.maximum(deg, 1.0))
    o_ref[...] = (x_ref[...] * r[:, None]).astype(jnp.bfloat16)


_xn_call = pl.pallas_call(
    _xn_body, out_shape=jax.ShapeDtypeStruct((N, D), jnp.bfloat16))


def _out_body(acc_ref, degB_ref, w_ref, b_ref, o_ref):
    deg = jnp.sum(degB_ref[...], axis=0)
    r = lax.rsqrt(jnp.maximum(deg, 1.0))
    S = (acc_ref[:N, :].astype(jnp.float32) +
         acc_ref[N:2 * N, :].astype(jnp.float32) +
         acc_ref[2 * N:3 * N, :].astype(jnp.float32) +
         acc_ref[3 * N:, :].astype(jnp.float32)) * r[:, None]
    y = jnp.dot(S, w_ref[...], preferred_element_type=jnp.float32)
    o_ref[...] = jnp.maximum(y + b_ref[...][None, :], 0.0)


_out_call = pl.pallas_call(
    _out_body, out_shape=jax.ShapeDtypeStruct((N, U), jnp.float32))


def kernel(X, ref_A, ref_B, W, b):
    degA_parts, degB_parts = _deg_kernel(ref_A, ref_B)
    Xn = _xn_call(degA_parts, X)
    acc = _agg_kernel(Xn, ref_A, ref_B)
    return _out_call(acc, degB_parts, W, b)


# speedup vs baseline: 1.0509x; 1.0012x over previous
import functools

import jax
import jax.numpy as jnp
from jax import lax
from jax.experimental import pallas as pl
from jax.experimental.pallas import tpu as pltpu
from jax.experimental.pallas import tpu_sc as plsc

N = 10000
E = 320000
D = 128
U = 128

NC = 2
NS = 16
NW = NC * NS
LANES = 16

EPW = E // NW
K = 128
FULL_CHUNKS = EPW // K
TAIL = EPW - FULL_CHUNKS * K
CH = 2000
RPT = N // NS

_mesh = plsc.VectorSubcoreMesh(core_axis_name="c", subcore_axis_name="s")
_sc_params = pltpu.CompilerParams(needs_layout_passes=False,
                                  use_tc_tiling_on_sc=False)


@functools.partial(
    pl.kernel,
    out_type=(
        jax.ShapeDtypeStruct((NW, N), jnp.float32),
        jax.ShapeDtypeStruct((NW, N), jnp.float32),
    ),
    mesh=_mesh,
    scratch_types=[
        pltpu.VMEM((N,), jnp.float32),
        pltpu.VMEM((N,), jnp.float32),
        [pltpu.VMEM((CH,), jnp.int32) for _ in range(2)],
        [pltpu.VMEM((CH,), jnp.int32) for _ in range(2)],
        [pltpu.SemaphoreType.DMA for _ in range(2)],
    ],
    compiler_params=_sc_params,
)
def _deg_kernel(refA_hbm, refB_hbm, outA, outB, histA, histB, bufA, bufB,
                dsem):
    c = lax.axis_index("c")
    s = lax.axis_index("s")
    wid = s * NC + c

    zeros = jnp.zeros((LANES,), jnp.float32)

    def zero_body(i, carry):
        histA[pl.ds(i * LANES, LANES)] = zeros
        histB[pl.ds(i * LANES, LANES)] = zeros
        return carry

    lax.fori_loop(0, N // LANES, zero_body, 0)

    base = wid * EPW
    ones = jnp.ones((LANES,), jnp.float32)
    NCHUNK = EPW // CH
    UNROLL = 5

    def load(o, b):
        pltpu.async_copy(refA_hbm.at[pl.ds(base + o * CH, CH)], bufA[b],
                         dsem[b])
        pltpu.async_copy(refB_hbm.at[pl.ds(base + o * CH, CH)], bufB[b],
                         dsem[b])

    load(0, 0)
    for o in range(NCHUNK):
        b = o % 2
        if o + 1 < NCHUNK:
            load(o + 1, 1 - b)
        pltpu.make_async_copy(refA_hbm.at[pl.ds(0, CH)], bufA[b],
                              dsem[b]).wait()
        pltpu.make_async_copy(refB_hbm.at[pl.ds(0, CH)], bufB[b],
                              dsem[b]).wait()

        def inner(i, icarry):
            for u in range(UNROLL):
                off = (i * UNROLL + u) * LANES
                ia = bufA[b][pl.ds(off, LANES)]
                ib = bufB[b][pl.ds(off, LANES)]
                plsc.addupdate_scatter(histA, [ia], ones)
                plsc.addupdate_scatter(histB, [ib], ones)
            return icarry

        lax.fori_loop(0, CH // (LANES * UNROLL), inner, 0)

    pltpu.sync_copy(histA, outA.at[wid])
    pltpu.sync_copy(histB, outB.at[wid])


NBUF = 4

@functools.partial(
    pl.kernel,
    out_type=jax.ShapeDtypeStruct((2 * NC * N, D), jnp.bfloat16),
    mesh=_mesh,
    scratch_types=[
        pltpu.VMEM((EPW,), jnp.int32),
        [pltpu.VMEM((K,), jnp.int32) for _ in range(NBUF)],
        pltpu.VMEM((TAIL,), jnp.int32),
        [pltpu.VMEM((K, D), jnp.bfloat16) for _ in range(NBUF)],
        [pltpu.VMEM_SHARED((N, D), jnp.bfloat16) for _ in range(2)],
        [pltpu.SemaphoreType.DMA for _ in range(NBUF)],
        [pltpu.SemaphoreType.DMA for _ in range(NBUF)],
        pltpu.SemaphoreType.DMA,
        pltpu.SemaphoreType.DMA,
    ],
    compiler_params=_sc_params,
)
def _agg_kernel(xn_hbm, refA_hbm, refB_hbm, out_hbm,
                idxA_full, idxB_s, idxB_t, rows, accs, gsem, isem, zsem, asem):
    c = lax.axis_index("c")
    s = lax.axis_index("s")

    ebase = (c * NS + s) * EPW
    zeros = jnp.zeros((2 * LANES,), jnp.bfloat16)

    def zrow(r, carry):
        for j in range(D // (2 * LANES)):
            rows[0][r, pl.ds(j * 2 * LANES, 2 * LANES)] = zeros
        return carry

    lax.fori_loop(0, K, zrow, 0)

    NZ = RPT // K
    ZREM = RPT - NZ * K
    for acc in accs:
        for j in range(NZ):
            pltpu.async_copy(rows[0].at[pl.ds(0, K)],
                             acc.at[pl.ds(s * RPT + j * K, K)], zsem)
        pltpu.async_copy(rows[0].at[pl.ds(0, ZREM)],
                         acc.at[pl.ds(s * RPT + NZ * K, ZREM)], zsem)
    pltpu.async_copy(refA_hbm.at[pl.ds(ebase, EPW)], idxA_full, asem)

    def load_idxB(o, b):
        pltpu.async_copy(refB_hbm.at[pl.ds(ebase + o * K, K)], idxB_s[b],
                         isem[b])

    for b in range(NBUF):
        load_idxB(b, b)

    for _ in range(2):
        for j in range(NZ):
            pltpu.make_async_copy(xn_hbm.at[pl.ds(0, K)],
                                  rows[0].at[pl.ds(0, K)], zsem).wait()
        pltpu.make_async_copy(xn_hbm.at[pl.ds(0, ZREM)],
                              rows[0].at[pl.ds(0, ZREM)], zsem).wait()
    plsc.subcore_barrier()
    pltpu.make_async_copy(refA_hbm.at[pl.ds(0, EPW)], idxA_full, asem).wait()

    def start_gather(o, b):
        pltpu.async_copy(xn_hbm.at[idxA_full.at[pl.ds(o * K, K)]],
                         rows[b], gsem[b])

    for b in range(NBUF):
        start_gather(b, b)

    def process(o, b):
        pltpu.make_async_copy(xn_hbm.at[pl.ds(0, K)], rows[b], gsem[b]).wait()
        pltpu.make_async_copy(refB_hbm.at[pl.ds(0, K)], idxB_s[b],
                              isem[b]).wait()
        pltpu.sync_copy(rows[b], accs[b % 2].at[idxB_s[b]], add=True)

    def group(g, carry):
        for b in range(NBUF):
            o = g * NBUF + b
            process(o, b)
            load_idxB(o + NBUF, b)
            start_gather(o + NBUF, b)
        return carry

    G = (FULL_CHUNKS - NBUF - 1) // NBUF
    lax.fori_loop(0, G, group, 0)
    tail_b = (FULL_CHUNKS - 2) % NBUF
    for o in range(G * NBUF, FULL_CHUNKS):
        b = o % NBUF
        process(o, b)
        if o + NBUF < FULL_CHUNKS:
            load_idxB(o + NBUF, b)
            start_gather(o + NBUF, b)
        if o == FULL_CHUNKS - 2:
            pltpu.async_copy(
                refB_hbm.at[pl.ds(ebase + FULL_CHUNKS * K, TAIL)], idxB_t,
                isem[tail_b])
            pltpu.async_copy(
                xn_hbm.at[idxA_full.at[pl.ds(FULL_CHUNKS * K, TAIL)]],
                rows[tail_b].at[pl.ds(0, TAIL)], gsem[tail_b])

    pltpu.make_async_copy(xn_hbm.at[pl.ds(0, TAIL)],
                          rows[tail_b].at[pl.ds(0, TAIL)],
                          gsem[tail_b]).wait()
    pltpu.make_async_copy(refB_hbm.at[pl.ds(0, TAIL)], idxB_t,
                          isem[tail_b]).wait()
    pltpu.sync_copy(rows[tail_b].at[pl.ds(0, TAIL)], accs[0].at[idxB_t],
                    add=True)

    plsc.subcore_barrier()
    for p in range(2):
        pltpu.sync_copy(
            accs[p].at[pl.ds(s * RPT, RPT)],
            out_hbm.at[pl.ds((2 * c + p) * N + s * RPT, RPT)])


def _xn_body(degA_ref, x_ref, o_ref):
    deg = jnp.sum(degA_ref[...], axis=0)
    r = lax.rsqrt(jnp.maximum(deg, 1.0))
    o_ref[...] = (x_ref[...] * r[:, None]).astype(jnp.bfloat16)


_xn_call = pl.pallas_call(
    _xn_body, out_shape=jax.ShapeDtypeStruct((N, D), jnp.bfloat16))


def _out_body(acc_ref, degB_ref, w_ref, b_ref, o_ref):
    deg = jnp.sum(degB_ref[...], axis=0)
    r = lax.rsqrt(jnp.maximum(deg, 1.0))
    S = ((acc_ref[:N, :] + acc_ref[N:2 * N, :]) +
         (acc_ref[2 * N:3 * N, :] + acc_ref[3 * N:, :]))
    y = jnp.dot(S, w_ref[...].astype(jnp.bfloat16),
                preferred_element_type=jnp.float32)
    o_ref[...] = jnp.maximum(y * r[:, None] + b_ref[...][None, :], 0.0)


_out_call = pl.pallas_call(
    _out_body, out_shape=jax.ShapeDtypeStruct((N, U), jnp.float32))


def kernel(X, ref_A, ref_B, W, b):
    degA_parts, degB_parts = _deg_kernel(ref_A, ref_B)
    Xn = _xn_call(degA_parts, X)
    acc = _agg_kernel(Xn, ref_A, ref_B)
    return _out_call(acc, degB_parts, W, b)


# speedup vs baseline: 1.1158x; 1.0618x over previous
import functools

import jax
import jax.numpy as jnp
from jax import lax
from jax.experimental import pallas as pl
from jax.experimental.pallas import tpu as pltpu
from jax.experimental.pallas import tpu_sc as plsc

N = 10000
E = 320000
D = 128
U = 128

NC = 2
NS = 16
NW = NC * NS
LANES = 16

EPW = E // NW
K = 128
FULL_CHUNKS = EPW // K
TAIL = EPW - FULL_CHUNKS * K
CH = 2000
RPT = N // NS

_mesh = plsc.VectorSubcoreMesh(core_axis_name="c", subcore_axis_name="s")
_sc_params = pltpu.CompilerParams(needs_layout_passes=False,
                                  use_tc_tiling_on_sc=False)


@functools.partial(
    pl.kernel,
    out_type=(
        jax.ShapeDtypeStruct((NW, N), jnp.float32),
        jax.ShapeDtypeStruct((NW, N), jnp.float32),
    ),
    mesh=_mesh,
    scratch_types=[
        pltpu.VMEM((N,), jnp.float32),
        pltpu.VMEM((N,), jnp.float32),
        [pltpu.VMEM((CH,), jnp.int32) for _ in range(2)],
        [pltpu.VMEM((CH,), jnp.int32) for _ in range(2)],
        [pltpu.SemaphoreType.DMA for _ in range(2)],
    ],
    compiler_params=_sc_params,
)
def _deg_kernel(refA_hbm, refB_hbm, outA, outB, histA, histB, bufA, bufB,
                dsem):
    c = lax.axis_index("c")
    s = lax.axis_index("s")
    wid = s * NC + c

    zeros = jnp.zeros((LANES,), jnp.float32)

    def zero_body(i, carry):
        histA[pl.ds(i * LANES, LANES)] = zeros
        histB[pl.ds(i * LANES, LANES)] = zeros
        return carry

    lax.fori_loop(0, N // LANES, zero_body, 0)

    base = wid * EPW
    ones = jnp.ones((LANES,), jnp.float32)
    NCHUNK = EPW // CH
    UNROLL = 5

    def load(o, b):
        pltpu.async_copy(refA_hbm.at[pl.ds(base + o * CH, CH)], bufA[b],
                         dsem[b])
        pltpu.async_copy(refB_hbm.at[pl.ds(base + o * CH, CH)], bufB[b],
                         dsem[b])

    load(0, 0)
    for o in range(NCHUNK):
        b = o % 2
        if o + 1 < NCHUNK:
            load(o + 1, 1 - b)
        pltpu.make_async_copy(refA_hbm.at[pl.ds(0, CH)], bufA[b],
                              dsem[b]).wait()
        pltpu.make_async_copy(refB_hbm.at[pl.ds(0, CH)], bufB[b],
                              dsem[b]).wait()

        def inner(i, icarry):
            for u in range(UNROLL):
                off = (i * UNROLL + u) * LANES
                ia = bufA[b][pl.ds(off, LANES)]
                ib = bufB[b][pl.ds(off, LANES)]
                plsc.addupdate_scatter(histA, [ia], ones)
                plsc.addupdate_scatter(histB, [ib], ones)
            return icarry

        lax.fori_loop(0, CH // (LANES * UNROLL), inner, 0)

    pltpu.sync_copy(histA, outA.at[wid])
    pltpu.sync_copy(histB, outB.at[wid])


NBUF = 2

@functools.partial(
    pl.kernel,
    out_type=jax.ShapeDtypeStruct((NC * N, D), jnp.float32),
    mesh=_mesh,
    scratch_types=[
        pltpu.VMEM((EPW,), jnp.int32),
        [pltpu.VMEM((K,), jnp.int32) for _ in range(NBUF)],
        pltpu.VMEM((TAIL,), jnp.int32),
        [pltpu.VMEM((K, D), jnp.float32) for _ in range(NBUF)],
        pltpu.VMEM_SHARED((N, D), jnp.float32),
        [pltpu.SemaphoreType.DMA for _ in range(NBUF)],
        [pltpu.SemaphoreType.DMA for _ in range(NBUF)],
        pltpu.SemaphoreType.DMA,
        pltpu.SemaphoreType.DMA,
    ],
    compiler_params=_sc_params,
)
def _agg_kernel(xn_hbm, refA_hbm, refB_hbm, out_hbm,
                idxA_full, idxB_s, idxB_t, rows, acc, gsem, isem, zsem, asem):
    c = lax.axis_index("c")
    s = lax.axis_index("s")

    ebase = (c * NS + s) * EPW
    zeros = jnp.zeros((LANES,), jnp.float32)

    def zrow(r, carry):
        for j in range(D // LANES):
            rows[0][r, pl.ds(j * LANES, LANES)] = zeros
        return carry

    lax.fori_loop(0, K, zrow, 0)

    NZ = RPT // K
    ZREM = RPT - NZ * K
    for j in range(NZ):
        pltpu.async_copy(rows[0].at[pl.ds(0, K)],
                         acc.at[pl.ds(s * RPT + j * K, K)], zsem)
    pltpu.async_copy(rows[0].at[pl.ds(0, ZREM)],
                     acc.at[pl.ds(s * RPT + NZ * K, ZREM)], zsem)
    pltpu.async_copy(refA_hbm.at[pl.ds(ebase, EPW)], idxA_full, asem)

    def load_idxB(o, b):
        pltpu.async_copy(refB_hbm.at[pl.ds(ebase + o * K, K)], idxB_s[b],
                         isem[b])

    for b in range(NBUF):
        load_idxB(b, b)

    for j in range(NZ):
        pltpu.make_async_copy(xn_hbm.at[pl.ds(0, K)], rows[0].at[pl.ds(0, K)],
                              zsem).wait()
    pltpu.make_async_copy(xn_hbm.at[pl.ds(0, ZREM)],
                          rows[0].at[pl.ds(0, ZREM)], zsem).wait()
    plsc.subcore_barrier()
    pltpu.make_async_copy(refA_hbm.at[pl.ds(0, EPW)], idxA_full, asem).wait()

    def start_gather(o, b):
        pltpu.async_copy(xn_hbm.at[idxA_full.at[pl.ds(o * K, K)]],
                         rows[b], gsem[b])

    for b in range(NBUF):
        start_gather(b, b)

    def process(o, b):
        pltpu.make_async_copy(xn_hbm.at[pl.ds(0, K)], rows[b], gsem[b]).wait()
        pltpu.make_async_copy(refB_hbm.at[pl.ds(0, K)], idxB_s[b],
                              isem[b]).wait()
        pltpu.sync_copy(rows[b], acc.at[idxB_s[b]], add=True)

    def group(g, carry):
        for b in range(NBUF):
            o = g * NBUF + b
            process(o, b)
            load_idxB(o + NBUF, b)
            start_gather(o + NBUF, b)
        return carry

    G = (FULL_CHUNKS - NBUF - 1) // NBUF
    lax.fori_loop(0, G, group, 0)
    tail_b = (FULL_CHUNKS - 2) % NBUF
    for o in range(G * NBUF, FULL_CHUNKS):
        b = o % NBUF
        process(o, b)
        if o + NBUF < FULL_CHUNKS:
            load_idxB(o + NBUF, b)
            start_gather(o + NBUF, b)
        if o == FULL_CHUNKS - 2:
            pltpu.async_copy(
                refB_hbm.at[pl.ds(ebase + FULL_CHUNKS * K, TAIL)], idxB_t,
                isem[tail_b])
            pltpu.async_copy(
                xn_hbm.at[idxA_full.at[pl.ds(FULL_CHUNKS * K, TAIL)]],
                rows[tail_b].at[pl.ds(0, TAIL)], gsem[tail_b])

    pltpu.make_async_copy(xn_hbm.at[pl.ds(0, TAIL)],
                          rows[tail_b].at[pl.ds(0, TAIL)],
                          gsem[tail_b]).wait()
    pltpu.make_async_copy(refB_hbm.at[pl.ds(0, TAIL)], idxB_t,
                          isem[tail_b]).wait()
    pltpu.sync_copy(rows[tail_b].at[pl.ds(0, TAIL)], acc.at[idxB_t], add=True)

    plsc.subcore_barrier()
    pltpu.sync_copy(acc.at[pl.ds(s * RPT, RPT)],
                    out_hbm.at[pl.ds(c * N + s * RPT, RPT)])


def _xn_body(degA_ref, x_ref, o_ref):
    deg = jnp.sum(degA_ref[...], axis=0)
    r = lax.rsqrt(jnp.maximum(deg, 1.0))
    o_ref[...] = x_ref[...] * r[:, None]


_xn_call = pl.pallas_call(
    _xn_body, out_shape=jax.ShapeDtypeStruct((N, D), jnp.float32))


def _out_body(acc_ref, degB_ref, w_ref, b_ref, o_ref):
    deg = jnp.sum(degB_ref[...], axis=0)
    r = lax.rsqrt(jnp.maximum(deg, 1.0))
    S = (acc_ref[:N, :] + acc_ref[N:, :]) * r[:, None]
    y = jnp.dot(S, w_ref[...], preferred_element_type=jnp.float32)
    o_ref[...] = jnp.maximum(y + b_ref[...][None, :], 0.0)


_out_call = pl.pallas_call(
    _out_body, out_shape=jax.ShapeDtypeStruct((N, U), jnp.float32))


def kernel(X, ref_A, ref_B, W, b):
    degA_parts, degB_parts = _deg_kernel(ref_A, ref_B)
    Xn = _xn_call(degA_parts, X)
    acc = _agg_kernel(Xn, ref_A, ref_B)
    return _out_call(acc, degB_parts, W, b)
